# baseline (speedup 1.0000x reference)
import jax, jax.numpy as jnp
from jax.experimental import pallas as pl

B, S, D = 4, 2048, 1024
BS = 512

def _tc_body(x_ref, w_ref, o_ref):
    o_ref[0] = x_ref[0] + w_ref[...]

_tc_call = pl.pallas_call(
    _tc_body,
    grid=(B, S // BS),
    in_specs=[
        pl.BlockSpec((1, BS, D), lambda b, i: (b, i, 0)),
        pl.BlockSpec((BS, D), lambda b, i: (i, 0)),
    ],
    out_specs=pl.BlockSpec((1, BS, D), lambda b, i: (b, i, 0)),
    out_shape=jax.ShapeDtypeStruct((B, S, D), jnp.float32),
)

def kernel(x, wpe):
    return _tc_call(x, wpe)
